# Initial kernel scaffold; baseline (speedup 1.0000x reference)
import jax
import jax.numpy as jnp
from jax.experimental import pallas as pl


def kernel(x, log_dt, A_real_log, A_imag, C_re, C_im, D, W_out, b_out):
    raise NotImplementedError("write your pallas kernel here")


# speedup vs baseline: 9.7610x; 9.7610x over previous
import functools

import jax
import jax.numpy as jnp
from jax.experimental import pallas as pl
from jax.experimental.pallas import tpu as pltpu


def _ssm_conv_kernel(ld_ref, d_ref, arl_ref, aim_ref, cre_ref, cim_ref,
                     x_ref, y_ref, *, ch, L, N):
    hb = pl.program_id(0)
    row = jax.lax.broadcasted_iota(jnp.int32, (L, L), 0)
    col = jax.lax.broadcasted_iota(jnp.int32, (L, L), 1)
    t = jax.lax.broadcasted_iota(jnp.int32, (L, N), 0).astype(jnp.float32)
    for c in range(ch):
        h = hb * ch + c
        dt = jnp.exp(ld_ref[h])
        dd = d_ref[h]
        a_re = -jnp.exp(arl_ref[c:c + 1, :])
        a_im = aim_ref[c:c + 1, :]
        da_re = a_re * dt
        da_im = a_im * dt
        E = jnp.exp(t * da_re)
        ph = t * da_im
        Cc = jnp.cos(ph)
        Ss = jnp.sin(ph)
        V_re = E * Cc
        V_im = E * Ss
        e_re = V_re[1:2, :]
        e_im = V_im[1:2, :]
        inv_a2 = 1.0 / (a_re * a_re + a_im * a_im)
        n_re = e_re - 1.0
        n_im = e_im
        q_re = (n_re * a_re + n_im * a_im) * inv_a2
        q_im = (n_im * a_re - n_re * a_im) * inv_a2
        c_re = cre_ref[c:c + 1, :]
        c_im = cim_ref[c:c + 1, :]
        cd_re = 2.0 * (c_re * q_re - c_im * q_im)
        cd_im = 2.0 * (c_re * q_im + c_im * q_re)
        P_re = cd_re * V_re - cd_im * V_im
        P_im = cd_re * V_im + cd_im * V_re
        Einv = 1.0 / E
        Q_re = Cc * Einv
        Q_im = -Ss * Einv
        dn = (((1,), (1,)), ((), ()))
        m_raw = (jax.lax.dot_general(Q_re, P_re, dn,
                                     preferred_element_type=jnp.float32)
                 - jax.lax.dot_general(Q_im, P_im, dn,
                                       preferred_element_type=jnp.float32))
        m_mat = jnp.where(col >= row, m_raw, 0.0)
        m_mat = jnp.where(col == row, m_mat + dd, m_mat)
        y = jnp.dot(x_ref[:, c, :], m_mat, preferred_element_type=jnp.float32)
        y_ref[:, c, :] = jax.nn.gelu(y)


def _glu_kernel(w_ref, b_ref, y1_ref, o_ref, *, H):
    y = y1_ref[0]
    a = jnp.dot(w_ref[:H, :], y, preferred_element_type=jnp.float32) \
        + b_ref[:H, :]
    g = jnp.dot(w_ref[H:, :], y, preferred_element_type=jnp.float32) \
        + b_ref[H:, :]
    o_ref[0] = a * jax.nn.sigmoid(g)


def kernel(x, log_dt, A_real_log, A_imag, C_re, C_im, D, W_out, b_out):
    B, H, L = x.shape
    N = A_imag.shape[1]
    CH = 8
    interp = False

    y1 = pl.pallas_call(
        functools.partial(_ssm_conv_kernel, ch=CH, L=L, N=N),
        out_shape=jax.ShapeDtypeStruct((B, H, L), jnp.float32),
        grid=(H // CH,),
        in_specs=[
            pl.BlockSpec(memory_space=pltpu.SMEM),
            pl.BlockSpec(memory_space=pltpu.SMEM),
            pl.BlockSpec((CH, N), lambda i: (i, 0)),
            pl.BlockSpec((CH, N), lambda i: (i, 0)),
            pl.BlockSpec((CH, N), lambda i: (i, 0)),
            pl.BlockSpec((CH, N), lambda i: (i, 0)),
            pl.BlockSpec((B, CH, L), lambda i: (0, i, 0)),
        ],
        out_specs=pl.BlockSpec((B, CH, L), lambda i: (0, i, 0)),
        compiler_params=pltpu.CompilerParams(
            dimension_semantics=("parallel",),
        ),
        name="s4d_conv",
        interpret=interp,
    )(log_dt, D, A_real_log, A_imag, C_re, C_im, x)

    bias2d = jnp.broadcast_to(b_out[:, None], (2 * H, L))
    out = pl.pallas_call(
        functools.partial(_glu_kernel, H=H),
        out_shape=jax.ShapeDtypeStruct((B, H, L), jnp.float32),
        grid=(B,),
        in_specs=[
            pl.BlockSpec((2 * H, H), lambda i: (0, 0)),
            pl.BlockSpec((2 * H, L), lambda i: (0, 0)),
            pl.BlockSpec((1, H, L), lambda i: (i, 0, 0)),
        ],
        out_specs=pl.BlockSpec((1, H, L), lambda i: (i, 0, 0)),
        compiler_params=pltpu.CompilerParams(
            dimension_semantics=("parallel",),
        ),
        name="glu_proj",
        interpret=interp,
    )(W_out, bias2d, y1)
    return out


# speedup vs baseline: 14.0166x; 1.4360x over previous
import functools

import jax
import jax.numpy as jnp
from jax.experimental import pallas as pl
from jax.experimental.pallas import tpu as pltpu


def _swap64(x):
    return jnp.concatenate([x[..., 64:], x[..., :64]], axis=-1)


def _cmul_pk(x, m, islo):
    swap_m = _swap64(m)
    a = jnp.where(islo, m, swap_m)
    b = jnp.where(islo, -swap_m, m)
    return x * a + _swap64(x) * b


def _build_pow(init8, mult8, islo1, L):
    x = init8
    m = mult8
    p = 8
    while p < L:
        x = jnp.concatenate([x, _cmul_pk(x, m, islo1)], axis=1)
        p *= 2
        if p < L:
            m = _cmul_pk(m, m, islo1)
    return x


def _ssm_conv_kernel(ld_ref, d_ref, arl_ref, aim_ref, cre_ref, cim_ref,
                     x_ref, y_ref, *, ch, L, N):
    f32 = jnp.float32
    row = jax.lax.broadcasted_iota(jnp.int32, (L, L), 0)
    col = jax.lax.broadcasted_iota(jnp.int32, (L, L), 1)
    islo1 = jax.lax.broadcasted_iota(jnp.int32, (ch, 1, 2 * N), 2) < N
    islo8 = jax.lax.broadcasted_iota(jnp.int32, (ch, 8, 2 * N), 2) < N
    tt8 = jax.lax.broadcasted_iota(jnp.int32, (ch, 8, 2 * N), 1).astype(f32)

    dt = jnp.exp(ld_ref[...])
    da_re = -jnp.exp(arl_ref[...]) * dt
    da_im = aim_ref[...] * dt

    e1 = jnp.exp(da_re)
    c1 = jnp.cos(da_im)
    s1 = jnp.sin(da_im)
    n_re = e1 * c1 - 1.0
    n_im = e1 * s1
    a_re = da_re / dt
    a_im = aim_ref[...]
    inv_a2 = 1.0 / (a_re * a_re + a_im * a_im)
    q_re = (n_re * a_re + n_im * a_im) * inv_a2
    q_im = (n_im * a_re - n_re * a_im) * inv_a2
    cd_re = 2.0 * (cre_ref[...] * q_re - cim_ref[...] * q_im)
    cd_im = 2.0 * (cre_ref[...] * q_im + cim_ref[...] * q_re)

    def pk1(re, im):
        return jnp.concatenate([re, im], axis=-1).reshape(ch, 1, 2 * N)

    da_re_dup = pk1(da_re, da_re)
    da_im_dup = pk1(da_im, da_im)
    cdr_dup = pk1(cd_re, cd_re)
    cdi_dup = pk1(cd_im, cd_im)

    E8 = jnp.exp(tt8 * da_re_dup)
    ph8 = tt8 * da_im_dup
    cos8 = jnp.cos(ph8)
    sin8 = jnp.sin(ph8)
    X8 = jnp.where(islo8, cos8, sin8)
    Y8 = jnp.where(islo8, -sin8, cos8)
    P8 = E8 * (cdr_dup * X8 + cdi_dup * Y8)
    G8 = X8 / E8

    e8 = jnp.exp(8.0 * da_re)
    c8 = jnp.cos(8.0 * da_im)
    s8 = jnp.sin(8.0 * da_im)
    w8 = pk1(e8 * c8, e8 * s8)
    einv8 = 1.0 / e8
    g8 = pk1(c8 * einv8, s8 * einv8)

    dn = (((1,), (1,)), ((), ()))
    gsz = 16
    for g in range(0, ch, gsz):
        sl = slice(g, g + gsz)
        islo_g = islo1[sl]
        P = _build_pow(P8[sl], w8[sl], islo_g, L)
        G = _build_pow(G8[sl], g8[sl], islo_g, L)
        for c in range(gsz):
            m_raw = jax.lax.dot_general(G[c], P[c], dn,
                                        preferred_element_type=f32)
            m_raw = jnp.where(col == row, m_raw + d_ref[g + c, 0], m_raw)
            m_mat = jnp.where(col >= row, m_raw, 0.0)
            y = jnp.dot(x_ref[:, g + c, :], m_mat,
                        preferred_element_type=f32)
            y_ref[g + c] = jax.nn.gelu(y).astype(jnp.bfloat16)


def _glu_kernel(w_ref, b_ref, y_ref, o_ref, *, H, L, bb):
    z = jnp.dot(w_ref[...], y_ref[...],
                preferred_element_type=jnp.float32) + b_ref[...]
    for j in range(bb):
        zj = z[:, j * L:(j + 1) * L]
        o_ref[j] = zj[:H, :] * jax.nn.sigmoid(zj[H:, :])


def kernel(x, log_dt, A_real_log, A_imag, C_re, C_im, D, W_out, b_out):
    B, H, L = x.shape
    N = A_imag.shape[1]
    CH = 64

    y1 = pl.pallas_call(
        functools.partial(_ssm_conv_kernel, ch=CH, L=L, N=N),
        out_shape=jax.ShapeDtypeStruct((H, B, L), jnp.bfloat16),
        grid=(H // CH,),
        in_specs=[
            pl.BlockSpec((CH, 1), lambda i: (i, 0)),
            pl.BlockSpec((CH, 1), lambda i: (i, 0)),
            pl.BlockSpec((CH, N), lambda i: (i, 0)),
            pl.BlockSpec((CH, N), lambda i: (i, 0)),
            pl.BlockSpec((CH, N), lambda i: (i, 0)),
            pl.BlockSpec((CH, N), lambda i: (i, 0)),
            pl.BlockSpec((B, CH, L), lambda i: (0, i, 0)),
        ],
        out_specs=pl.BlockSpec((CH, B, L), lambda i: (i, 0, 0)),
        compiler_params=pltpu.CompilerParams(
            dimension_semantics=("parallel",),
        ),
        name="s4d_conv",
    )(log_dt[:, None], D[:, None], A_real_log, A_imag, C_re, C_im, x)

    y1v = y1.reshape(H, B * L)
    w_bf = W_out.astype(jnp.bfloat16)
    BB = 4
    bias2d = jnp.broadcast_to(b_out[:, None, None], (2 * H, BB, L))
    bias2d = bias2d.reshape(2 * H, BB * L)
    out = pl.pallas_call(
        functools.partial(_glu_kernel, H=H, L=L, bb=BB),
        out_shape=jax.ShapeDtypeStruct((B, H, L), jnp.float32),
        grid=(B // BB,),
        in_specs=[
            pl.BlockSpec((2 * H, H), lambda i: (0, 0)),
            pl.BlockSpec((2 * H, BB * L), lambda i: (0, 0)),
            pl.BlockSpec((H, BB * L), lambda i: (0, i)),
        ],
        out_specs=pl.BlockSpec((BB, H, L), lambda i: (i, 0, 0)),
        compiler_params=pltpu.CompilerParams(
            dimension_semantics=("parallel",),
        ),
        name="glu_proj",
    )(w_bf, bias2d, y1v)
    return out
